# baseline (speedup 1.0000x reference)
import jax
import jax.numpy as jnp
from jax import lax
from jax.experimental import pallas as pl
from jax.experimental.pallas import tpu as pltpu
from jax.experimental.pallas import tpu_sc as plsc

D = 64
S = 200
B = 4096
V = 1000000
NC = 2
NS = 16
NW = NC * NS
SCALE = 8.0

BW = B // NW
NBUF = 4
OBUF = 2
PE_S = 208
PE_D = 128
RSTR = 128


def _encoder_body(xt_hbm, tokp_hbm, pes_hbm, out_hbm,
                  xv_v, idx_v, rows_v, obuf_v, hov_v, pec_v, gsem, osem, isem, psem):
    wid = lax.axis_index("s") * NC + lax.axis_index("c")
    col0 = wid * BW

    lanes = lax.iota(jnp.int32, 16)

    def issue_xv(c, b):
        pltpu.async_copy(xt_hbm.at[c, pl.ds(col0, BW)], xv_v.at[b],
                         isem.at[b])
        pltpu.async_copy(pes_hbm.at[c], pec_v.at[b], psem.at[b])

    def wait_xv(b):
        pltpu.make_async_copy(xt_hbm.at[0, pl.ds(col0, BW)], xv_v.at[b],
                              isem.at[b]).wait()
        pltpu.make_async_copy(pes_hbm.at[0], pec_v.at[b], psem.at[b]).wait()

    def issue_gather(b):
        @pl.loop(0, BW // 16)
        def _(g):
            sl = pl.ds(g * 16, 16)
            idx_v[b, sl] = lax.shift_right_logical(xv_v[b, sl], 1)
        pltpu.async_copy(tokp_hbm.at[idx_v.at[b]],
                         rows_v.at[b, :, pl.ds(0, 128)], gsem.at[b])

    def wait_gather(b):
        pltpu.make_async_copy(tokp_hbm.at[idx_v.at[b]],
                              rows_v.at[b, :, pl.ds(0, 128)],
                              gsem.at[b]).wait()

    def compute(c, b):
        ob = b % OBUF

        @pl.loop(0, BW // 16)
        def _(g):
            sl = pl.ds(g * 16, 16)
            hov_v[sl] = lax.shift_left(lax.bitwise_and(xv_v[b, sl], 1), 6)

        @pl.loop(0, BW // 16)
        def _(g):
            sl = pl.ds(g * 16, 16)
            ho = hov_v[sl]
            bidx = lanes + g * 16
            for d in range(D):
                val = plsc.load_gather(rows_v.at[b], [bidx, ho + d])
                pe_d = pec_v[b, pl.ds(d * 16, 16)]
                obuf_v[ob, d, sl] = val * SCALE + pe_d

    def issue_scatter(c, b):
        ob = b % OBUF
        pltpu.async_copy(obuf_v.at[ob],
                         out_hbm.at[c, :, pl.ds(col0, BW)], osem.at[ob])

    def wait_scatter(c, ob):
        pltpu.make_async_copy(obuf_v.at[ob],
                              out_hbm.at[0, :, pl.ds(col0, BW)],
                              osem.at[ob]).wait()

    for b in range(NBUF - 1):
        issue_xv(b, b)
        wait_xv(b)
        issue_gather(b)
    issue_xv(NBUF - 1, NBUF - 1)

    @pl.loop(0, S, step=NBUF)
    def _chunks(c0):
        for b in range(NBUF):
            c = c0 + b
            prev = (b - 1) % NBUF

            @pl.when(c + NBUF - 1 < S)
            def _():
                wait_xv(prev)
                issue_gather(prev)

            wait_gather(b)

            @pl.when(c >= OBUF)
            def _():
                wait_scatter(c - OBUF, b % OBUF)

            compute(c, b)
            issue_scatter(c, b)

            @pl.when(c + NBUF < S)
            def _():
                issue_xv(c + NBUF, b)

    for c in range(S - OBUF, S):
        wait_scatter(c, c % OBUF)


@jax.jit
def _encoder(xt, tokp, pes):
    mesh = plsc.VectorSubcoreMesh(core_axis_name="c", subcore_axis_name="s")
    return pl.kernel(
        _encoder_body,
        out_type=jax.ShapeDtypeStruct((S, D, B), jnp.float32),
        mesh=mesh,
        compiler_params=pltpu.CompilerParams(use_tc_tiling_on_sc=True,
                                             needs_layout_passes=False,
                                             disable_bounds_checks=True),
        scratch_types=[
            pltpu.VMEM((NBUF, BW), jnp.int32),
            pltpu.VMEM((NBUF, BW), jnp.int32),
            pltpu.VMEM((NBUF, BW, RSTR), jnp.float32),
            pltpu.VMEM((OBUF, D, BW), jnp.float32),
            pltpu.VMEM((BW,), jnp.int32),
            pltpu.VMEM((NBUF, D * 16), jnp.float32),
            pltpu.SemaphoreType.DMA((NBUF,)),
            pltpu.SemaphoreType.DMA((OBUF,)),
            pltpu.SemaphoreType.DMA((NBUF,)),
            pltpu.SemaphoreType.DMA((NBUF,)),
        ],
    )(xt, tokp, pes)


def kernel(x, mask, tok_table, pos_table):
    del mask
    xt = jnp.transpose(x.astype(jnp.int32))
    tokp = jnp.reshape(tok_table, (V // 2, 128))
    pes = jnp.reshape(jnp.broadcast_to(pos_table[:, :, None], (S, D, 16)),
                      (S, D * 16))
    out_t = _encoder(xt, tokp, pes)
    return jnp.transpose(out_t, (2, 0, 1))


# speedup vs baseline: 1.7617x; 1.7617x over previous
import functools

import jax
import jax.numpy as jnp
from jax import lax
from jax.experimental import pallas as pl
from jax.experimental.pallas import tpu as pltpu
from jax.experimental.pallas import tpu_sc as plsc

D = 64
S = 200
B = 4096
NC = 2
NS = 16
NW = NC * NS
SCALE = 8.0

SEQ_PER_CHUNK = 2
CHUNK = SEQ_PER_CHUNK * S
SUB = 100
NSUB = CHUNK // SUB
NBUF = 4
ROWS_PER_W = (B * S) // NW
N_CHUNKS = ROWS_PER_W // CHUNK


def _encoder_body(x_hbm, tok_hbm, pos_hbm, out_hbm,
                  idx_v, rows_v, pe_v, gsem, osem, isem):
    wid = lax.axis_index("s") * NC + lax.axis_index("c")
    out_base = wid * ROWS_PER_W

    pltpu.sync_copy(pos_hbm, pe_v)

    def issue_gathers(c, b):
        for j in range(NSUB):
            pltpu.async_copy(
                tok_hbm.at[idx_v.at[b, j]],
                rows_v.at[b, pl.ds(j * SUB, SUB)],
                gsem.at[b],
            )

    def wait_gathers(b):
        for j in range(NSUB):
            pltpu.make_async_copy(
                tok_hbm.at[idx_v.at[b, j]],
                rows_v.at[b, pl.ds(j * SUB, SUB)],
                gsem.at[b],
            ).wait()

    def issue_idx_load(c, b):
        pltpu.async_copy(x_hbm.at[wid, c], idx_v.at[b], isem.at[b])

    def wait_idx_load(b):
        pltpu.make_async_copy(
            x_hbm.at[wid, 0], idx_v.at[b], isem.at[b]
        ).wait()

    def compute_chunk(b):
        @pl.loop(0, S)
        def _per_position(s):
            for d in range(D // 16):
                sl = pl.ds(d * 16, 16)
                pe_d = pe_v[s, sl]
                for j in range(SEQ_PER_CHUNK):
                    r = j * S + s
                    rows_v[b, r, sl] = rows_v[b, r, sl] * SCALE + pe_d

    def issue_scatter(c, b):
        pltpu.async_copy(
            rows_v.at[b],
            out_hbm.at[pl.ds(out_base + c * CHUNK, CHUNK)],
            osem.at[b],
        )

    def wait_scatter(c, b):
        pltpu.make_async_copy(
            rows_v.at[b],
            out_hbm.at[pl.ds(out_base + c * CHUNK, CHUNK)],
            osem.at[b],
        ).wait()

    for b in range(NBUF - 1):
        issue_idx_load(b, b)
        wait_idx_load(b)
        issue_gathers(b, b)
    issue_idx_load(NBUF - 1, NBUF - 1)

    @pl.loop(0, N_CHUNKS, step=NBUF)
    def _chunk_group(c0):
        for b in range(NBUF):
            c = c0 + b
            prev = (b - 1) % NBUF

            @pl.when(c > 0)
            def _():
                wait_scatter(c - 1, prev)

            @pl.when(c + NBUF - 1 < N_CHUNKS)
            def _():
                wait_idx_load(prev)
                issue_gathers(c + NBUF - 1, prev)

            wait_gathers(b)

            @pl.when(c + NBUF < N_CHUNKS)
            def _():
                issue_idx_load(c + NBUF, b)

            compute_chunk(b)
            issue_scatter(c, b)

    wait_scatter(N_CHUNKS - 1, (N_CHUNKS - 1) % NBUF)


@jax.jit
def _encoder(x_r, tok_table, pos_table):
    mesh = plsc.VectorSubcoreMesh(core_axis_name="c", subcore_axis_name="s")
    return pl.kernel(
        _encoder_body,
        out_type=jax.ShapeDtypeStruct((B * S, D), jnp.float32),
        mesh=mesh,
        compiler_params=pltpu.CompilerParams(use_tc_tiling_on_sc=False),
        scratch_types=[
            pltpu.VMEM((NBUF, NSUB, SUB), jnp.int32),
            pltpu.VMEM((NBUF, CHUNK, D), jnp.float32),
            pltpu.VMEM((S, D), jnp.float32),
            pltpu.SemaphoreType.DMA((NBUF,)),
            pltpu.SemaphoreType.DMA((NBUF,)),
            pltpu.SemaphoreType.DMA((NBUF,)),
        ],
    )(x_r, tok_table, pos_table)


def kernel(x, mask, tok_table, pos_table):
    del mask
    x_r = x.astype(jnp.int32).reshape(NW, N_CHUNKS, NSUB, SUB)
    out = _encoder(x_r, tok_table, pos_table)
    return out.reshape(B, S, D)
